# baseline (speedup 1.0000x reference)
import functools

import jax
import jax.numpy as jnp
from jax.experimental import pallas as pl
from jax.experimental.pallas import tpu as pltpu

_PACK = 8
_HID = 32
_SLOT = 16
_LANE = 128


def _round_up(x: int, m: int) -> int:
    return ((x + m - 1) // m) * m


def _packed_mlp_kernel(x_ref, w1_ref, b1_ref, w2_ref, b2_ref, w3_ref, b3_ref,
                       o_ref):
    h = jnp.dot(x_ref[...], w1_ref[...], preferred_element_type=jnp.float32)
    h = jnp.maximum(h + b1_ref[...], 0.0)
    h = jnp.dot(h, w2_ref[...], preferred_element_type=jnp.float32)
    h = jnp.maximum(h + b2_ref[...], 0.0)
    o = jnp.dot(h, w3_ref[...], preferred_element_type=jnp.float32)
    o_ref[...] = (o + b3_ref[...]).astype(o_ref.dtype)


def kernel(x, w1, b1, w2, b2, w3, b3):
    B, in_dim = x.shape
    out_dim = w3.shape[1]

    w1s = w1[:, :_HID]
    w2s = w2[:_HID, :_HID]
    w3s = w3[:_HID, :]
    b1s, b2s = b1[:, :_HID], b2[:, :_HID]

    eye = jnp.eye(_PACK, dtype=x.dtype)
    w1e = jnp.pad(w1s, ((0, _SLOT - in_dim), (0, 0)))
    w3e = jnp.pad(w3s, ((0, 0), (0, _SLOT - out_dim)))
    w1p = jnp.einsum("ab,ij->aibj", eye, w1e).reshape(_PACK * _SLOT,
                                                      _PACK * _HID)
    w2p = jnp.einsum("ab,ij->aibj", eye, w2s).reshape(_PACK * _HID,
                                                      _PACK * _HID)
    w3p = jnp.einsum("ab,ij->aibj", eye, w3e).reshape(_PACK * _HID,
                                                      _PACK * _SLOT)
    b1p = jnp.tile(b1s, (1, _PACK))
    b2p = jnp.tile(b2s, (1, _PACK))
    b3e = jnp.pad(b3, ((0, 0), (0, _SLOT - out_dim)))
    b3p = jnp.tile(b3e, (1, _PACK))

    bp = _round_up(B, _PACK * 8)
    xp = x if bp == B else jnp.zeros((bp, in_dim), x.dtype).at[:B].set(x)
    m = bp // _PACK
    x16 = jnp.pad(xp.reshape(m, _PACK, in_dim),
                  ((0, 0), (0, 0), (0, _SLOT - in_dim))).reshape(m, _LANE)

    tb = 8192
    while m % tb:
        tb //= 2
    grid = (m // tb,)

    const = lambda i: (0, 0)
    out = pl.pallas_call(
        _packed_mlp_kernel,
        out_shape=jax.ShapeDtypeStruct((m, _LANE), x.dtype),
        grid=grid,
        in_specs=[
            pl.BlockSpec((tb, _LANE), lambda i: (i, 0)),
            pl.BlockSpec(w1p.shape, const),
            pl.BlockSpec(b1p.shape, const),
            pl.BlockSpec(w2p.shape, const),
            pl.BlockSpec(b2p.shape, const),
            pl.BlockSpec(w3p.shape, const),
            pl.BlockSpec(b3p.shape, const),
        ],
        out_specs=pl.BlockSpec((tb, _LANE), lambda i: (i, 0)),
        compiler_params=pltpu.CompilerParams(
            dimension_semantics=("parallel",)),
        name="snake_mlp_packed8",
    )(x16, w1p, b1p, w2p, b2p, w3p, b3p)

    o = out.reshape(m, _PACK, _SLOT)[:, :, :out_dim].reshape(bp, out_dim)
    return o[:B]


# speedup vs baseline: 22.7278x; 22.7278x over previous
import functools

import jax
import jax.numpy as jnp
from jax.experimental import pallas as pl
from jax.experimental.pallas import tpu as pltpu

_HID = 32
_LANE = 128


def _round_up(x: int, m: int) -> int:
    return ((x + m - 1) // m) * m


def _snake_t_kernel(x_ref, w1_ref, b1_ref, w2_ref, b2_ref, w3_ref, b3_ref,
                    o_ref):
    h = jnp.dot(w1_ref[...], x_ref[...], preferred_element_type=jnp.float32)
    h = jnp.maximum(h + b1_ref[...], 0.0)
    h = jnp.dot(w2_ref[...], h, preferred_element_type=jnp.float32)
    h = jnp.maximum(h + b2_ref[...], 0.0)
    o = jnp.dot(w3_ref[...], h, preferred_element_type=jnp.float32)
    o_ref[...] = (o + b3_ref[...]).astype(o_ref.dtype)


def kernel(x, w1, b1, w2, b2, w3, b3):
    B, in_dim = x.shape
    out_dim = w3.shape[1]

    w1t = w1[:, :_HID].T
    w2t = w2[:_HID, :_HID].T
    w3t = w3[:_HID, :].T
    b1t = b1[:, :_HID].T
    b2t = b2[:, :_HID].T
    b3t = b3.T

    xt = x.T

    bp = _round_up(B, _LANE)
    if bp != B:
        xt = jnp.zeros((in_dim, bp), x.dtype).at[:, :B].set(xt)
    nb = 32768
    while bp % nb:
        nb //= 2
    grid = (bp // nb,)

    const = lambda i: (0, 0)
    out = pl.pallas_call(
        _snake_t_kernel,
        out_shape=jax.ShapeDtypeStruct((out_dim, bp), x.dtype),
        grid=grid,
        in_specs=[
            pl.BlockSpec((in_dim, nb), lambda i: (0, i)),
            pl.BlockSpec(w1t.shape, const),
            pl.BlockSpec(b1t.shape, const),
            pl.BlockSpec(w2t.shape, const),
            pl.BlockSpec(b2t.shape, const),
            pl.BlockSpec(w3t.shape, const),
            pl.BlockSpec(b3t.shape, const),
        ],
        out_specs=pl.BlockSpec((out_dim, nb), lambda i: (0, i)),
        compiler_params=pltpu.CompilerParams(
            dimension_semantics=("parallel",)),
        name="snake_mlp_t",
    )(xt, w1t, b1t, w2t, b2t, w3t, b3t)

    return out[:, :B].T


# speedup vs baseline: 24.5750x; 1.0813x over previous
import functools

import jax
import jax.numpy as jnp
from jax.experimental import pallas as pl
from jax.experimental.pallas import tpu as pltpu

_HID = 32
_LANE = 128


def _round_up(x: int, m: int) -> int:
    return ((x + m - 1) // m) * m


def _snake_t_kernel(x_ref, w1_ref, b1_ref, w2_ref, b2_ref, w3_ref, b3_ref,
                    o_ref):
    h = jnp.dot(w1_ref[...], x_ref[...], preferred_element_type=jnp.float32)
    h = jnp.maximum(h + b1_ref[...], 0.0)
    h = jnp.dot(w2_ref[...], h, preferred_element_type=jnp.float32)
    h = jnp.maximum(h + b2_ref[...], 0.0)
    o = jnp.dot(w3_ref[...], h, preferred_element_type=jnp.float32)
    o_ref[...] = (o + b3_ref[...]).astype(o_ref.dtype)


def kernel(x, w1, b1, w2, b2, w3, b3):
    B, in_dim = x.shape
    out_dim = w3.shape[1]

    w1t = w1[:, :_HID].T
    w2t = w2[:_HID, :_HID].T
    w3t = w3[:_HID, :].T
    b1t = b1[:, :_HID].T
    b2t = b2[:, :_HID].T
    b3t = b3.T

    xt = x.T

    bp = _round_up(B, _LANE)
    if bp != B:
        xt = jnp.zeros((in_dim, bp), x.dtype).at[:, :B].set(xt)
    nb = 65536
    while bp % nb:
        nb //= 2
    grid = (bp // nb,)

    const = lambda i: (0, 0)
    out = pl.pallas_call(
        _snake_t_kernel,
        out_shape=jax.ShapeDtypeStruct((out_dim, bp), x.dtype),
        grid=grid,
        in_specs=[
            pl.BlockSpec((in_dim, nb), lambda i: (0, i)),
            pl.BlockSpec(w1t.shape, const),
            pl.BlockSpec(b1t.shape, const),
            pl.BlockSpec(w2t.shape, const),
            pl.BlockSpec(b2t.shape, const),
            pl.BlockSpec(w3t.shape, const),
            pl.BlockSpec(b3t.shape, const),
        ],
        out_specs=pl.BlockSpec((out_dim, nb), lambda i: (0, i)),
        compiler_params=pltpu.CompilerParams(
            dimension_semantics=("parallel",)),
        name="snake_mlp_t",
    )(xt, w1t, b1t, w2t, b2t, w3t, b3t)

    return out[:, :B].T
